# Initial kernel scaffold; baseline (speedup 1.0000x reference)
import jax
import jax.numpy as jnp
from jax.experimental import pallas as pl


def kernel(x, edge_index, emb, W_src1, W_dst1, att_src1, att_dst1, bias1, W_src2, W_dst2, att_src2, att_dst2, bias2):
    raise NotImplementedError("write your pallas kernel here")


# speedup vs baseline: 1.1557x; 1.1557x over previous
import jax
import jax.numpy as jnp
from jax.experimental import pallas as pl

N = 50000
E = 800000
HID = 64
OUT = 32
HEADS = 4


def _gat(h_src, h_dst, src, dst, Ws, Wd, a_s, a_d, b, H, C):
    n = h_dst.shape[0]
    xs = (h_src @ Ws).reshape(-1, H, C)
    xd = (h_dst @ Wd).reshape(-1, H, C)
    a_src = jnp.sum(xs * a_s, axis=-1)
    a_dst = jnp.sum(xd * a_d, axis=-1)
    alpha = jax.nn.leaky_relu(a_src[src] + a_dst[dst], 0.2)
    e = jnp.exp(alpha)
    ssum = jax.ops.segment_sum(e, dst, num_segments=n)
    msg = xs[src] * e[:, :, None]
    out = jax.ops.segment_sum(msg, dst, num_segments=n)
    out = out / (ssum[:, :, None] + 1e-16)
    return out.reshape(n, H * C) + b


def kernel(x, edge_index, emb, W_src1, W_dst1, att_src1, att_dst1, bias1,
           W_src2, W_dst2, att_src2, att_dst2, bias2):
    src = edge_index[0]
    dst = edge_index[1]
    h = emb
    h1 = jax.nn.relu(_gat(h, h, src, dst, W_src1, W_dst1, att_src1, att_dst1, bias1, HEADS, HID // HEADS))
    h2 = jax.nn.relu(_gat(h1, h1, src, dst, W_src2, W_dst2, att_src2, att_dst2, bias2, 1, OUT))
    return h2


# speedup vs baseline: 33.5426x; 29.0240x over previous
import jax
import jax.numpy as jnp
from jax import lax
from jax.experimental import pallas as pl
from jax.experimental.pallas import tpu as pltpu
from jax.experimental.pallas import tpu_sc as plsc

N = 50000
E = 800000
HID = 64
OUT = 32
HEADS = 4

_B = 128
_NCHUNKS = E // _B
_NW = 32
_NA = 50048
_RPT = _NA // 16
_NP = 50176

_SC_PARAMS = pltpu.CompilerParams(
    needs_layout_passes=False, use_tc_tiling_on_sc=False)
_MESH = dict(core_axis_name="c", subcore_axis_name="s")


def _make_edge_a(H):
    def body(at_hbm, src_hbm, dst_hbm, zeros_hbm, e_hbm, ssum_hbm,
             srcb, dstb, ats, atd, ebuf, stage, ssum_sh, sem, sem2):
        cid = lax.axis_index("c")
        sid = lax.axis_index("s")
        w = sid * 2 + cid
        r0 = sid * _RPT

        z16 = jnp.zeros((16,), jnp.float32)
        for j in range(_B // 16):
            ridx0 = lax.iota(jnp.int32, 16) + (j * 16)
            for c in range(H, 8):
                plsc.store_scatter(
                    stage, [ridx0, jnp.full((16,), c, jnp.int32)], z16)

        pltpu.sync_copy(zeros_hbm.at[pl.ds(r0, _RPT)],
                        ssum_sh.at[pl.ds(r0, _RPT)])
        plsc.subcore_barrier()

        def chunk(i, carry):
            c = w + _NW * i
            base = c * _B
            pltpu.sync_copy(src_hbm.at[pl.ds(base, _B)], srcb)
            pltpu.sync_copy(dst_hbm.at[pl.ds(base, _B)], dstb)
            cs = pltpu.async_copy(at_hbm.at[srcb], ats, sem)
            cd = pltpu.async_copy(at_hbm.at[dstb], atd, sem2)
            cs.wait()
            cd.wait()
            for j in range(_B // 16):
                ridx = lax.iota(jnp.int32, 16) + (j * 16)
                for h in range(H):
                    av = plsc.load_gather(
                        ats, [ridx, jnp.full((16,), h, jnp.int32)])
                    bv = plsc.load_gather(
                        atd, [ridx, jnp.full((16,), 4 + h, jnp.int32)])
                    s = av + bv
                    e = jnp.exp(jnp.where(s >= 0, s, 0.2 * s))
                    ebuf[h, pl.ds(j * 16, 16)] = e
                    plsc.store_scatter(
                        stage, [ridx, jnp.full((16,), h, jnp.int32)], e)
            pltpu.sync_copy(ebuf, e_hbm.at[c])
            pltpu.sync_copy(stage, ssum_sh.at[dstb], add=True)
            return carry

        nmy = (_NCHUNKS - w + _NW - 1) // _NW
        lax.fori_loop(0, nmy, chunk, 0)
        plsc.subcore_barrier()
        pltpu.sync_copy(ssum_sh.at[pl.ds(r0, _RPT)],
                        ssum_hbm.at[cid, pl.ds(r0, _RPT)])
        plsc.subcore_barrier()

    return pl.kernel(
        body,
        out_type=(
            jax.ShapeDtypeStruct((_NCHUNKS, H, _B), jnp.float32),
            jax.ShapeDtypeStruct((2, _NA, 8), jnp.float32),
        ),
        mesh=plsc.VectorSubcoreMesh(**_MESH),
        scratch_types=[
            pltpu.VMEM((_B,), jnp.int32),
            pltpu.VMEM((_B,), jnp.int32),
            pltpu.VMEM((_B, 8), jnp.float32),
            pltpu.VMEM((_B, 8), jnp.float32),
            pltpu.VMEM((H, _B), jnp.float32),
            pltpu.VMEM((_B, 8), jnp.float32),
            pltpu.VMEM_SHARED((_NA, 8), jnp.float32),
            pltpu.SemaphoreType.DMA,
            pltpu.SemaphoreType.DMA,
        ],
        compiler_params=_SC_PARAMS,
    )


def _make_edge_b(H, C):
    def body(xv_hbm, e_hbm, src_hbm, dst_hbm, zeros_hbm, out_hbm,
             srcb, dstb, idxb, ech, rows, acc_sh, sem):
        cid = lax.axis_index("c")
        sid = lax.axis_index("s")
        w = sid * 2 + cid
        r0 = sid * _RPT

        for h in range(H):
            pltpu.sync_copy(zeros_hbm.at[pl.ds(r0, _RPT)],
                            acc_sh.at[pl.ds(r0, _RPT)])
            plsc.subcore_barrier()

            def chunk(i, carry):
                c = w + _NW * i
                base = c * _B
                pltpu.sync_copy(src_hbm.at[pl.ds(base, _B)], srcb)
                pltpu.sync_copy(dst_hbm.at[pl.ds(base, _B)], dstb)
                pltpu.sync_copy(e_hbm.at[c, h], ech)
                if H > 1:
                    for j in range(_B // 16):
                        sl = pl.ds(j * 16, 16)
                        idxb[sl] = srcb[sl] * H + h
                    gref = idxb
                else:
                    gref = srcb
                pltpu.async_copy(xv_hbm.at[gref], rows, sem).wait()
                for j in range(_B // 16):
                    e16 = ech[pl.ds(j * 16, 16)]
                    for i16 in range(16):
                        k = j * 16 + i16
                        eb = e16.at[jnp.full((16,), i16, jnp.int32)].get(
                            mode="promise_in_bounds")
                        for c0 in range(0, C, 16):
                            rows[k, pl.ds(c0, 16)] = (
                                rows[k, pl.ds(c0, 16)] * eb)
                pltpu.sync_copy(rows, acc_sh.at[dstb], add=True)
                return carry

            nmy = (_NCHUNKS - w + _NW - 1) // _NW
            lax.fori_loop(0, nmy, chunk, 0)
            plsc.subcore_barrier()
            pltpu.sync_copy(acc_sh.at[pl.ds(r0, _RPT)],
                            out_hbm.at[h, cid, pl.ds(r0, _RPT)])
            plsc.subcore_barrier()

    return pl.kernel(
        body,
        out_type=jax.ShapeDtypeStruct((H, 2, _NA, C), jnp.float32),
        mesh=plsc.VectorSubcoreMesh(**_MESH),
        scratch_types=[
            pltpu.VMEM((_B,), jnp.int32),
            pltpu.VMEM((_B,), jnp.int32),
            pltpu.VMEM((_B,), jnp.int32),
            pltpu.VMEM((_B,), jnp.float32),
            pltpu.VMEM((_B, C), jnp.float32),
            pltpu.VMEM_SHARED((_NA, C), jnp.float32),
            pltpu.SemaphoreType.DMA,
        ],
        compiler_params=_SC_PARAMS,
    )


_edge_a1 = _make_edge_a(HEADS)
_edge_a2 = _make_edge_a(1)
_edge_b1 = _make_edge_b(HEADS, HID // HEADS)
_edge_b2 = _make_edge_b(1, OUT)

_NB = 2000
_NB1 = 1792


def _prep1_body(h_ref, w_ref, acat_ref, x_ref, at_ref):
    hb = h_ref[...]
    x_ref[...] = jnp.dot(hb, w_ref[...], preferred_element_type=jnp.float32)
    at_ref[...] = jnp.dot(hb, acat_ref[...],
                          preferred_element_type=jnp.float32)


_prep1 = pl.pallas_call(
    _prep1_body,
    grid=(_NP // _NB1,),
    in_specs=[
        pl.BlockSpec((_NB1, HID), lambda i: (i, 0)),
        pl.BlockSpec((HID, HID), lambda i: (0, 0)),
        pl.BlockSpec((HID, 8), lambda i: (0, 0)),
    ],
    out_specs=[
        pl.BlockSpec((_NB1, HID), lambda i: (i, 0)),
        pl.BlockSpec((_NB1, 8), lambda i: (i, 0)),
    ],
    out_shape=[
        jax.ShapeDtypeStruct((_NP, HID), jnp.float32),
        jax.ShapeDtypeStruct((_NP, 8), jnp.float32),
    ],
)


def _fin1_body(acc_ref, ss_ref, b1_ref, w2_ref, acat2_ref, x2_ref, at2_ref):
    acc = acc_ref[...]
    a = acc[:, 0] + acc[:, 1]
    ss = ss_ref[...]
    s = ss[0] + ss[1]
    cols = []
    for h in range(HEADS):
        cols.append(a[h] / (s[:, h:h + 1] + 1e-16))
    h1 = jnp.concatenate(cols, axis=1) + b1_ref[...]
    h1 = jnp.maximum(h1, 0.0)
    x2_ref[...] = jnp.dot(h1, w2_ref[...], preferred_element_type=jnp.float32)
    at2_ref[...] = jnp.dot(h1, acat2_ref[...],
                           preferred_element_type=jnp.float32)


_fin1 = pl.pallas_call(
    _fin1_body,
    grid=(N // _NB,),
    in_specs=[
        pl.BlockSpec((HEADS, 2, _NB, 16), lambda i: (0, 0, i, 0)),
        pl.BlockSpec((2, _NB, 8), lambda i: (0, i, 0)),
        pl.BlockSpec((1, HID), lambda i: (0, 0)),
        pl.BlockSpec((HID, OUT), lambda i: (0, 0)),
        pl.BlockSpec((HID, 8), lambda i: (0, 0)),
    ],
    out_specs=[
        pl.BlockSpec((_NB, OUT), lambda i: (i, 0)),
        pl.BlockSpec((_NB, 8), lambda i: (i, 0)),
    ],
    out_shape=[
        jax.ShapeDtypeStruct((N, OUT), jnp.float32),
        jax.ShapeDtypeStruct((N, 8), jnp.float32),
    ],
)


def _fin2_body(acc_ref, ss_ref, b2_ref, out_ref):
    acc = acc_ref[...]
    a = acc[0] + acc[1]
    ss = ss_ref[...]
    s = ss[0] + ss[1]
    out_ref[...] = jnp.maximum(
        a / (s[:, 0:1] + 1e-16) + b2_ref[...], 0.0)


_fin2 = pl.pallas_call(
    _fin2_body,
    grid=(N // _NB,),
    in_specs=[
        pl.BlockSpec((2, _NB, OUT), lambda i: (0, i, 0)),
        pl.BlockSpec((2, _NB, 8), lambda i: (0, i, 0)),
        pl.BlockSpec((1, OUT), lambda i: (0, 0)),
    ],
    out_specs=pl.BlockSpec((_NB, OUT), lambda i: (i, 0)),
    out_shape=jax.ShapeDtypeStruct((N, OUT), jnp.float32),
)


def kernel(x, edge_index, emb, W_src1, W_dst1, att_src1, att_dst1, bias1,
           W_src2, W_dst2, att_src2, att_dst2, bias2):
    del x
    src = edge_index[0]
    dst = edge_index[1]
    C1 = HID // HEADS

    As1 = (W_src1.reshape(HID, HEADS, C1) * att_src1).sum(-1)
    Ad1 = (W_dst1.reshape(HID, HEADS, C1) * att_dst1).sum(-1)
    acat1 = jnp.concatenate([As1, Ad1], axis=1)
    As2 = (W_src2.reshape(HID, 1, OUT) * att_src2).sum(-1)
    Ad2 = (W_dst2.reshape(HID, 1, OUT) * att_dst2).sum(-1)
    pad3 = jnp.zeros((HID, 3), jnp.float32)
    acat2 = jnp.concatenate([As2, pad3, Ad2, pad3], axis=1)

    zeros8 = jnp.zeros((_NA, 8), jnp.float32)
    zeros1 = jnp.zeros((_NA, C1), jnp.float32)
    zeros2 = jnp.zeros((_NA, OUT), jnp.float32)
    embp = jnp.pad(emb, ((0, _NP - N), (0, 0)))

    x1, at1 = _prep1(embp, W_src1, acat1)
    e1, ss1 = _edge_a1(at1, src, dst, zeros8)
    acc1 = _edge_b1(x1.reshape(_NP * HEADS, C1), e1, src, dst, zeros1)
    x2, at2 = _fin1(acc1, ss1, bias1.reshape(1, HID), W_src2, acat2)
    e2, ss2 = _edge_a2(at2, src, dst, zeros8)
    acc2 = _edge_b2(x2, e2, src, dst, zeros2)
    h2 = _fin2(acc2.reshape(2, _NA, OUT), ss2, bias2.reshape(1, OUT))
    return h2


# speedup vs baseline: 53.1995x; 1.5860x over previous
import jax
import jax.numpy as jnp
from jax import lax
from jax.experimental import pallas as pl
from jax.experimental.pallas import tpu as pltpu
from jax.experimental.pallas import tpu_sc as plsc

N = 50000
E = 800000
HID = 64
OUT = 32
HEADS = 4

_B = 128
_NW = 32
_KPW = 196
_NCH = _NW * _KPW
_EP = _NCH * _B
_NA = 50048
_RPT = _NA // 16
_NP = 50176

_SC_PARAMS = pltpu.CompilerParams(
    needs_layout_passes=False, use_tc_tiling_on_sc=False)
_MESH = dict(core_axis_name="c", subcore_axis_name="s")


def _make_edge_a(H):
    def body(at_hbm, edges_hbm, zeros_hbm, e_hbm, ssum_hbm,
             eb0, eb1, ats0, ats1, atd0, atd1, ebuf, stage, ssum_sh,
             ss0, ss1, sd0, sd1):
        cid = lax.axis_index("c")
        sid = lax.axis_index("s")
        w = sid * 2 + cid
        r0 = sid * _RPT
        eb = (eb0, eb1)
        ats = (ats0, ats1)
        atd = (atd0, atd1)
        ssem = (ss0, ss1)
        dsem = (sd0, sd1)

        z16 = jnp.zeros((16,), jnp.float32)
        for j in range(_B // 16):
            ridx0 = lax.iota(jnp.int32, 16) + (j * 16)
            for c in range(H, 8):
                plsc.store_scatter(
                    stage, [ridx0, jnp.full((16,), c, jnp.int32)], z16)

        pltpu.sync_copy(zeros_hbm.at[pl.ds(r0, _RPT)],
                        ssum_sh.at[pl.ds(r0, _RPT)])
        plsc.subcore_barrier()

        for b in (0, 1):
            pltpu.sync_copy(edges_hbm.at[w + _NW * b], eb[b])
            pltpu.async_copy(at_hbm.at[eb[b].at[0]], ats[b], ssem[b])
            pltpu.async_copy(at_hbm.at[eb[b].at[1]], atd[b], dsem[b])

        def pair(i, carry):
            for b in (0, 1):
                k = 2 * i + b
                c = w + _NW * k
                pltpu.make_async_copy(
                    at_hbm.at[eb[b].at[0]], ats[b], ssem[b]).wait()
                pltpu.make_async_copy(
                    at_hbm.at[eb[b].at[1]], atd[b], dsem[b]).wait()
                for j in range(_B // 16):
                    ridx = lax.iota(jnp.int32, 16) + (j * 16)
                    for h in range(H):
                        av = plsc.load_gather(
                            ats[b], [ridx, jnp.full((16,), h, jnp.int32)])
                        bv = plsc.load_gather(
                            atd[b], [ridx, jnp.full((16,), 4 + h, jnp.int32)])
                        s = av + bv
                        e = jnp.exp(jnp.where(s >= 0, s, 0.2 * s))
                        ebuf[h, pl.ds(j * 16, 16)] = e
                        plsc.store_scatter(
                            stage, [ridx, jnp.full((16,), h, jnp.int32)], e)
                pltpu.sync_copy(ebuf, e_hbm.at[c])
                pltpu.sync_copy(stage, ssum_sh.at[eb[b].at[1]], add=True)
                kn = jnp.minimum(k + 2, _KPW - 1)
                pltpu.sync_copy(edges_hbm.at[w + _NW * kn], eb[b])
                pltpu.async_copy(at_hbm.at[eb[b].at[0]], ats[b], ssem[b])
                pltpu.async_copy(at_hbm.at[eb[b].at[1]], atd[b], dsem[b])
            return carry

        lax.fori_loop(0, _KPW // 2, pair, 0)
        for b in (0, 1):
            pltpu.make_async_copy(
                at_hbm.at[eb[b].at[0]], ats[b], ssem[b]).wait()
            pltpu.make_async_copy(
                at_hbm.at[eb[b].at[1]], atd[b], dsem[b]).wait()
        plsc.subcore_barrier()
        pltpu.sync_copy(ssum_sh.at[pl.ds(r0, _RPT)],
                        ssum_hbm.at[cid, pl.ds(r0, _RPT)])
        plsc.subcore_barrier()

    return pl.kernel(
        body,
        out_type=(
            jax.ShapeDtypeStruct((_NCH, H, _B), jnp.float32),
            jax.ShapeDtypeStruct((2, _NA, 8), jnp.float32),
        ),
        mesh=plsc.VectorSubcoreMesh(**_MESH),
        scratch_types=[
            pltpu.VMEM((2, _B), jnp.int32),
            pltpu.VMEM((2, _B), jnp.int32),
            pltpu.VMEM((_B, 8), jnp.float32),
            pltpu.VMEM((_B, 8), jnp.float32),
            pltpu.VMEM((_B, 8), jnp.float32),
            pltpu.VMEM((_B, 8), jnp.float32),
            pltpu.VMEM((H, _B), jnp.float32),
            pltpu.VMEM((_B, 8), jnp.float32),
            pltpu.VMEM_SHARED((_NA, 8), jnp.float32),
            pltpu.SemaphoreType.DMA,
            pltpu.SemaphoreType.DMA,
            pltpu.SemaphoreType.DMA,
            pltpu.SemaphoreType.DMA,
        ],
        compiler_params=_SC_PARAMS,
    )


def _make_edge_b(H, C):
    def body(xv_hbm, e_hbm, edges_hbm, zeros_hbm, out_hbm,
             eb0, eb1, ix0, ix1, ec0, ec1, rw0, rw1, acc_sh, sg0, sg1):
        cid = lax.axis_index("c")
        sid = lax.axis_index("s")
        w = sid * 2 + cid
        r0 = sid * _RPT
        eb = (eb0, eb1)
        ixb = (ix0, ix1)
        ech = (ec0, ec1)
        rows = (rw0, rw1)
        gsem = (sg0, sg1)

        def load_and_gather(b, k, h):
            c = w + _NW * k
            pltpu.sync_copy(edges_hbm.at[c], eb[b])
            pltpu.sync_copy(e_hbm.at[c, h], ech[b])
            if H > 1:
                for j in range(_B // 16):
                    sl = pl.ds(j * 16, 16)
                    ixb[b][sl] = eb[b][0, sl] * H + h
                gref = ixb[b]
            else:
                gref = eb[b].at[0]
            pltpu.async_copy(xv_hbm.at[gref], rows[b], gsem[b])
            return gref

        def gwait(b):
            gref = ixb[b] if H > 1 else eb[b].at[0]
            pltpu.make_async_copy(xv_hbm.at[gref], rows[b], gsem[b]).wait()

        for h in range(H):
            pltpu.sync_copy(zeros_hbm.at[pl.ds(r0, _RPT)],
                            acc_sh.at[pl.ds(r0, _RPT)])
            plsc.subcore_barrier()

            for b in (0, 1):
                load_and_gather(b, b, h)

            def pair(i, carry):
                for b in (0, 1):
                    k = 2 * i + b
                    gwait(b)
                    for j in range(_B // 16):
                        e16 = ech[b][pl.ds(j * 16, 16)]
                        for i16 in range(16):
                            kk = j * 16 + i16
                            ev = e16.at[
                                jnp.full((16,), i16, jnp.int32)].get(
                                    mode="promise_in_bounds")
                            for c0 in range(0, C, 16):
                                rows[b][kk, pl.ds(c0, 16)] = (
                                    rows[b][kk, pl.ds(c0, 16)] * ev)
                    pltpu.sync_copy(rows[b], acc_sh.at[eb[b].at[1]],
                                    add=True)
                    kn = jnp.minimum(k + 2, _KPW - 1)
                    load_and_gather(b, kn, h)
                return carry

            lax.fori_loop(0, _KPW // 2, pair, 0)
            for b in (0, 1):
                gwait(b)
            plsc.subcore_barrier()
            pltpu.sync_copy(acc_sh.at[pl.ds(r0, _RPT)],
                            out_hbm.at[h, cid, pl.ds(r0, _RPT)])
            plsc.subcore_barrier()

    return pl.kernel(
        body,
        out_type=jax.ShapeDtypeStruct((H, 2, _NA, C), jnp.float32),
        mesh=plsc.VectorSubcoreMesh(**_MESH),
        scratch_types=[
            pltpu.VMEM((2, _B), jnp.int32),
            pltpu.VMEM((2, _B), jnp.int32),
            pltpu.VMEM((_B,), jnp.int32),
            pltpu.VMEM((_B,), jnp.int32),
            pltpu.VMEM((_B,), jnp.float32),
            pltpu.VMEM((_B,), jnp.float32),
            pltpu.VMEM((_B, C), jnp.float32),
            pltpu.VMEM((_B, C), jnp.float32),
            pltpu.VMEM_SHARED((_NA, C), jnp.float32),
            pltpu.SemaphoreType.DMA,
            pltpu.SemaphoreType.DMA,
        ],
        compiler_params=_SC_PARAMS,
    )


_edge_a1 = _make_edge_a(HEADS)
_edge_a2 = _make_edge_a(1)
_edge_b1 = _make_edge_b(HEADS, HID // HEADS)
_edge_b2 = _make_edge_b(1, OUT)

_NB = 2000
_NB1 = 1792


def _prep1_body(h_ref, w_ref, acat_ref, x_ref, at_ref):
    hb = h_ref[...]
    x_ref[...] = jnp.dot(hb, w_ref[...], preferred_element_type=jnp.float32)
    at_ref[...] = jnp.dot(hb, acat_ref[...],
                          preferred_element_type=jnp.float32)


_prep1 = pl.pallas_call(
    _prep1_body,
    grid=(_NP // _NB1,),
    in_specs=[
        pl.BlockSpec((_NB1, HID), lambda i: (i, 0)),
        pl.BlockSpec((HID, HID), lambda i: (0, 0)),
        pl.BlockSpec((HID, 8), lambda i: (0, 0)),
    ],
    out_specs=[
        pl.BlockSpec((_NB1, HID), lambda i: (i, 0)),
        pl.BlockSpec((_NB1, 8), lambda i: (i, 0)),
    ],
    out_shape=[
        jax.ShapeDtypeStruct((_NP, HID), jnp.float32),
        jax.ShapeDtypeStruct((_NP, 8), jnp.float32),
    ],
)


def _fin1_body(acc_ref, ss_ref, b1_ref, w2_ref, acat2_ref, x2_ref, at2_ref):
    acc = acc_ref[...]
    a = acc[:, 0] + acc[:, 1]
    ss = ss_ref[...]
    s = ss[0] + ss[1]
    cols = []
    for h in range(HEADS):
        cols.append(a[h] / (s[:, h:h + 1] + 1e-16))
    h1 = jnp.concatenate(cols, axis=1) + b1_ref[...]
    h1 = jnp.maximum(h1, 0.0)
    x2_ref[...] = jnp.dot(h1, w2_ref[...], preferred_element_type=jnp.float32)
    at2_ref[...] = jnp.dot(h1, acat2_ref[...],
                           preferred_element_type=jnp.float32)


_fin1 = pl.pallas_call(
    _fin1_body,
    grid=(N // _NB,),
    in_specs=[
        pl.BlockSpec((HEADS, 2, _NB, 16), lambda i: (0, 0, i, 0)),
        pl.BlockSpec((2, _NB, 8), lambda i: (0, i, 0)),
        pl.BlockSpec((1, HID), lambda i: (0, 0)),
        pl.BlockSpec((HID, OUT), lambda i: (0, 0)),
        pl.BlockSpec((HID, 8), lambda i: (0, 0)),
    ],
    out_specs=[
        pl.BlockSpec((_NB, OUT), lambda i: (i, 0)),
        pl.BlockSpec((_NB, 8), lambda i: (i, 0)),
    ],
    out_shape=[
        jax.ShapeDtypeStruct((N, OUT), jnp.float32),
        jax.ShapeDtypeStruct((N, 8), jnp.float32),
    ],
)


def _fin2_body(acc_ref, ss_ref, b2_ref, out_ref):
    acc = acc_ref[...]
    a = acc[0] + acc[1]
    ss = ss_ref[...]
    s = ss[0] + ss[1]
    out_ref[...] = jnp.maximum(
        a / (s[:, 0:1] + 1e-16) + b2_ref[...], 0.0)


_fin2 = pl.pallas_call(
    _fin2_body,
    grid=(N // _NB,),
    in_specs=[
        pl.BlockSpec((2, _NB, OUT), lambda i: (0, i, 0)),
        pl.BlockSpec((2, _NB, 8), lambda i: (0, i, 0)),
        pl.BlockSpec((1, OUT), lambda i: (0, 0)),
    ],
    out_specs=pl.BlockSpec((_NB, OUT), lambda i: (i, 0)),
    out_shape=jax.ShapeDtypeStruct((N, OUT), jnp.float32),
)


def kernel(x, edge_index, emb, W_src1, W_dst1, att_src1, att_dst1, bias1,
           W_src2, W_dst2, att_src2, att_dst2, bias2):
    del x
    src = edge_index[0]
    dst = edge_index[1]
    C1 = HID // HEADS

    As1 = (W_src1.reshape(HID, HEADS, C1) * att_src1).sum(-1)
    Ad1 = (W_dst1.reshape(HID, HEADS, C1) * att_dst1).sum(-1)
    acat1 = jnp.concatenate([As1, Ad1], axis=1)
    As2 = (W_src2.reshape(HID, 1, OUT) * att_src2).sum(-1)
    Ad2 = (W_dst2.reshape(HID, 1, OUT) * att_dst2).sum(-1)
    pad3 = jnp.zeros((HID, 3), jnp.float32)
    acat2 = jnp.concatenate([As2, pad3, Ad2, pad3], axis=1)

    zeros8 = jnp.zeros((_NA, 8), jnp.float32)
    zeros1 = jnp.zeros((_NA, C1), jnp.float32)
    zeros2 = jnp.zeros((_NA, OUT), jnp.float32)
    embp = jnp.pad(emb, ((0, _NP - N), (0, 0)))

    srcp = jnp.concatenate([src, jnp.zeros((_EP - E,), jnp.int32)])
    dstp = jnp.concatenate([dst, jnp.full((_EP - E,), N, jnp.int32)])
    edges_cm = jnp.stack(
        [srcp.reshape(_NCH, _B), dstp.reshape(_NCH, _B)], axis=1)

    x1, at1 = _prep1(embp, W_src1, acat1)
    e1, ss1 = _edge_a1(at1, edges_cm, zeros8)
    acc1 = _edge_b1(x1.reshape(_NP * HEADS, C1), e1, edges_cm, zeros1)
    x2, at2 = _fin1(acc1, ss1, bias1.reshape(1, HID), W_src2, acat2)
    e2, ss2 = _edge_a2(at2, edges_cm, zeros8)
    acc2 = _edge_b2(x2, e2, edges_cm, zeros2)
    h2 = _fin2(acc2.reshape(2, _NA, OUT), ss2, bias2.reshape(1, OUT))
    return h2


# speedup vs baseline: 62.8422x; 1.1813x over previous
import jax
import jax.numpy as jnp
from jax import lax
from jax.experimental import pallas as pl
from jax.experimental.pallas import tpu as pltpu
from jax.experimental.pallas import tpu_sc as plsc

N = 50000
E = 800000
HID = 64
OUT = 32
HEADS = 4

_B = 128
_NW = 32
_KPW = 196
_NCH = _NW * _KPW
_EP = _NCH * _B
_NA = 50048
_RPT = _NA // 16
_NP = 50176

_SC_PARAMS = pltpu.CompilerParams(
    needs_layout_passes=False, use_tc_tiling_on_sc=False)
_MESH = dict(core_axis_name="c", subcore_axis_name="s")


def _make_edge_a(H):
    def body(at_hbm, edges_hbm, zeros_hbm, e_hbm, ssum_hbm,
             eb0, eb1, ats0, ats1, atd0, atd1, ebuf, stage, ssum_sh,
             ss0, ss1, sd0, sd1):
        cid = lax.axis_index("c")
        sid = lax.axis_index("s")
        w = sid * 2 + cid
        r0 = sid * _RPT
        eb = (eb0, eb1)
        ats = (ats0, ats1)
        atd = (atd0, atd1)
        ssem = (ss0, ss1)
        dsem = (sd0, sd1)

        z16 = jnp.zeros((16,), jnp.float32)
        for j in range(_B // 16):
            ridx0 = lax.iota(jnp.int32, 16) + (j * 16)
            for c in range(H, 8):
                plsc.store_scatter(
                    stage, [ridx0, jnp.full((16,), c, jnp.int32)], z16)

        pltpu.sync_copy(zeros_hbm.at[pl.ds(r0, _RPT)],
                        ssum_sh.at[pl.ds(r0, _RPT)])
        plsc.subcore_barrier()

        for b in (0, 1):
            pltpu.sync_copy(edges_hbm.at[w + _NW * b], eb[b])
            pltpu.async_copy(at_hbm.at[eb[b].at[0]], ats[b], ssem[b])
            pltpu.async_copy(at_hbm.at[eb[b].at[1]], atd[b], dsem[b])

        def pair(i, carry):
            for b in (0, 1):
                k = 2 * i + b
                c = w + _NW * k
                pltpu.make_async_copy(
                    at_hbm.at[eb[b].at[0]], ats[b], ssem[b]).wait()
                pltpu.make_async_copy(
                    at_hbm.at[eb[b].at[1]], atd[b], dsem[b]).wait()
                for j in range(_B // 16):
                    sl = pl.ds(j * 16, 16)
                    ridx = lax.iota(jnp.int32, 16) + (j * 16)
                    s16 = eb[b][0, sl]
                    d16 = eb[b][1, sl]
                    for h in range(H):
                        av = plsc.load_gather(
                            ats[b], [ridx, jnp.full((16,), h, jnp.int32)])
                        bv = plsc.load_gather(
                            atd[b], [ridx, jnp.full((16,), 4 + h, jnp.int32)])
                        s = av + bv
                        e = jnp.exp(jnp.where(s >= 0, s, 0.2 * s))
                        ebuf[h, 0, sl] = plsc.bitcast(e, jnp.int32)
                        ebuf[h, 1, sl] = s16 * H + h
                        ebuf[h, 2, sl] = d16
                        plsc.store_scatter(
                            stage, [ridx, jnp.full((16,), h, jnp.int32)], e)
                pltpu.sync_copy(ebuf, e_hbm.at[c])
                pltpu.sync_copy(stage, ssum_sh.at[eb[b].at[1]], add=True)
                kn = jnp.minimum(k + 2, _KPW - 1)
                pltpu.sync_copy(edges_hbm.at[w + _NW * kn], eb[b])
                pltpu.async_copy(at_hbm.at[eb[b].at[0]], ats[b], ssem[b])
                pltpu.async_copy(at_hbm.at[eb[b].at[1]], atd[b], dsem[b])
            return carry

        lax.fori_loop(0, _KPW // 2, pair, 0)
        for b in (0, 1):
            pltpu.make_async_copy(
                at_hbm.at[eb[b].at[0]], ats[b], ssem[b]).wait()
            pltpu.make_async_copy(
                at_hbm.at[eb[b].at[1]], atd[b], dsem[b]).wait()
        plsc.subcore_barrier()
        pltpu.sync_copy(ssum_sh.at[pl.ds(r0, _RPT)],
                        ssum_hbm.at[cid, pl.ds(r0, _RPT)])
        plsc.subcore_barrier()

    return pl.kernel(
        body,
        out_type=(
            jax.ShapeDtypeStruct((_NCH, H, 3, _B), jnp.int32),
            jax.ShapeDtypeStruct((2, _NA, 8), jnp.float32),
        ),
        mesh=plsc.VectorSubcoreMesh(**_MESH),
        scratch_types=[
            pltpu.VMEM((2, _B), jnp.int32),
            pltpu.VMEM((2, _B), jnp.int32),
            pltpu.VMEM((_B, 8), jnp.float32),
            pltpu.VMEM((_B, 8), jnp.float32),
            pltpu.VMEM((_B, 8), jnp.float32),
            pltpu.VMEM((_B, 8), jnp.float32),
            pltpu.VMEM((H, 3, _B), jnp.int32),
            pltpu.VMEM((_B, 8), jnp.float32),
            pltpu.VMEM_SHARED((_NA, 8), jnp.float32),
            pltpu.SemaphoreType.DMA,
            pltpu.SemaphoreType.DMA,
            pltpu.SemaphoreType.DMA,
            pltpu.SemaphoreType.DMA,
        ],
        compiler_params=_SC_PARAMS,
    )


def _make_edge_b(H, C):
    def body(xv_hbm, rec_hbm, zeros_hbm, out_hbm,
             rb0, rb1, rw0, rw1, acc_sh, sg0, sg1):
        cid = lax.axis_index("c")
        sid = lax.axis_index("s")
        w = sid * 2 + cid
        r0 = sid * _RPT
        recb = (rb0, rb1)
        rows = (rw0, rw1)
        gsem = (sg0, sg1)

        def load_and_gather(b, k, h):
            c = w + _NW * k
            pltpu.sync_copy(rec_hbm.at[c, h], recb[b])
            pltpu.async_copy(xv_hbm.at[recb[b].at[1]], rows[b], gsem[b])

        def gwait(b):
            pltpu.make_async_copy(
                xv_hbm.at[recb[b].at[1]], rows[b], gsem[b]).wait()

        for h in range(H):
            pltpu.sync_copy(zeros_hbm.at[pl.ds(r0, _RPT)],
                            acc_sh.at[pl.ds(r0, _RPT)])
            plsc.subcore_barrier()

            for b in (0, 1):
                load_and_gather(b, b, h)

            def pair(i, carry):
                for b in (0, 1):
                    k = 2 * i + b
                    gwait(b)
                    for j in range(_B // 16):
                        e16 = plsc.bitcast(
                            recb[b][0, pl.ds(j * 16, 16)], jnp.float32)
                        for i16 in range(16):
                            kk = j * 16 + i16
                            ev = e16.at[
                                jnp.full((16,), i16, jnp.int32)].get(
                                    mode="promise_in_bounds")
                            for c0 in range(0, C, 16):
                                rows[b][kk, pl.ds(c0, 16)] = (
                                    rows[b][kk, pl.ds(c0, 16)] * ev)
                    pltpu.sync_copy(rows[b], acc_sh.at[recb[b].at[2]],
                                    add=True)
                    kn = jnp.minimum(k + 2, _KPW - 1)
                    load_and_gather(b, kn, h)
                return carry

            lax.fori_loop(0, _KPW // 2, pair, 0)
            for b in (0, 1):
                gwait(b)
            plsc.subcore_barrier()
            pltpu.sync_copy(acc_sh.at[pl.ds(r0, _RPT)],
                            out_hbm.at[h, cid, pl.ds(r0, _RPT)])
            plsc.subcore_barrier()

    return pl.kernel(
        body,
        out_type=jax.ShapeDtypeStruct((H, 2, _NA, C), jnp.float32),
        mesh=plsc.VectorSubcoreMesh(**_MESH),
        scratch_types=[
            pltpu.VMEM((3, _B), jnp.int32),
            pltpu.VMEM((3, _B), jnp.int32),
            pltpu.VMEM((_B, C), jnp.float32),
            pltpu.VMEM((_B, C), jnp.float32),
            pltpu.VMEM_SHARED((_NA, C), jnp.float32),
            pltpu.SemaphoreType.DMA,
            pltpu.SemaphoreType.DMA,
        ],
        compiler_params=_SC_PARAMS,
    )


_edge_a1 = _make_edge_a(HEADS)
_edge_a2 = _make_edge_a(1)
_edge_b1 = _make_edge_b(HEADS, HID // HEADS)
_edge_b2 = _make_edge_b(1, OUT)

_NB = 2000
_NB1 = 1792


def _prep1_body(h_ref, w_ref, acat_ref, x_ref, at_ref):
    hb = h_ref[...]
    x_ref[...] = jnp.dot(hb, w_ref[...], preferred_element_type=jnp.float32)
    at_ref[...] = jnp.dot(hb, acat_ref[...],
                          preferred_element_type=jnp.float32)


_prep1 = pl.pallas_call(
    _prep1_body,
    grid=(_NP // _NB1,),
    in_specs=[
        pl.BlockSpec((_NB1, HID), lambda i: (i, 0)),
        pl.BlockSpec((HID, HID), lambda i: (0, 0)),
        pl.BlockSpec((HID, 8), lambda i: (0, 0)),
    ],
    out_specs=[
        pl.BlockSpec((_NB1, HID), lambda i: (i, 0)),
        pl.BlockSpec((_NB1, 8), lambda i: (i, 0)),
    ],
    out_shape=[
        jax.ShapeDtypeStruct((_NP, HID), jnp.float32),
        jax.ShapeDtypeStruct((_NP, 8), jnp.float32),
    ],
)


def _fin1_body(acc_ref, ss_ref, b1_ref, w2_ref, acat2_ref, x2_ref, at2_ref):
    acc = acc_ref[...]
    a = acc[:, 0] + acc[:, 1]
    ss = ss_ref[...]
    s = ss[0] + ss[1]
    cols = []
    for h in range(HEADS):
        cols.append(a[h] / (s[:, h:h + 1] + 1e-16))
    h1 = jnp.concatenate(cols, axis=1) + b1_ref[...]
    h1 = jnp.maximum(h1, 0.0)
    x2_ref[...] = jnp.dot(h1, w2_ref[...], preferred_element_type=jnp.float32)
    at2_ref[...] = jnp.dot(h1, acat2_ref[...],
                           preferred_element_type=jnp.float32)


_fin1 = pl.pallas_call(
    _fin1_body,
    grid=(N // _NB,),
    in_specs=[
        pl.BlockSpec((HEADS, 2, _NB, 16), lambda i: (0, 0, i, 0)),
        pl.BlockSpec((2, _NB, 8), lambda i: (0, i, 0)),
        pl.BlockSpec((1, HID), lambda i: (0, 0)),
        pl.BlockSpec((HID, OUT), lambda i: (0, 0)),
        pl.BlockSpec((HID, 8), lambda i: (0, 0)),
    ],
    out_specs=[
        pl.BlockSpec((_NB, OUT), lambda i: (i, 0)),
        pl.BlockSpec((_NB, 8), lambda i: (i, 0)),
    ],
    out_shape=[
        jax.ShapeDtypeStruct((N, OUT), jnp.float32),
        jax.ShapeDtypeStruct((N, 8), jnp.float32),
    ],
)


def _fin2_body(acc_ref, ss_ref, b2_ref, out_ref):
    acc = acc_ref[...]
    a = acc[0] + acc[1]
    ss = ss_ref[...]
    s = ss[0] + ss[1]
    out_ref[...] = jnp.maximum(
        a / (s[:, 0:1] + 1e-16) + b2_ref[...], 0.0)


_fin2 = pl.pallas_call(
    _fin2_body,
    grid=(N // _NB,),
    in_specs=[
        pl.BlockSpec((2, _NB, OUT), lambda i: (0, i, 0)),
        pl.BlockSpec((2, _NB, 8), lambda i: (0, i, 0)),
        pl.BlockSpec((1, OUT), lambda i: (0, 0)),
    ],
    out_specs=pl.BlockSpec((_NB, OUT), lambda i: (i, 0)),
    out_shape=jax.ShapeDtypeStruct((N, OUT), jnp.float32),
)


def kernel(x, edge_index, emb, W_src1, W_dst1, att_src1, att_dst1, bias1,
           W_src2, W_dst2, att_src2, att_dst2, bias2):
    del x
    src = edge_index[0]
    dst = edge_index[1]
    C1 = HID // HEADS

    As1 = (W_src1.reshape(HID, HEADS, C1) * att_src1).sum(-1)
    Ad1 = (W_dst1.reshape(HID, HEADS, C1) * att_dst1).sum(-1)
    acat1 = jnp.concatenate([As1, Ad1], axis=1)
    As2 = (W_src2.reshape(HID, 1, OUT) * att_src2).sum(-1)
    Ad2 = (W_dst2.reshape(HID, 1, OUT) * att_dst2).sum(-1)
    pad3 = jnp.zeros((HID, 3), jnp.float32)
    acat2 = jnp.concatenate([As2, pad3, Ad2, pad3], axis=1)

    zeros8 = jnp.zeros((_NA, 8), jnp.float32)
    zeros1 = jnp.zeros((_NA, C1), jnp.float32)
    zeros2 = jnp.zeros((_NA, OUT), jnp.float32)
    embp = jnp.pad(emb, ((0, _NP - N), (0, 0)))

    srcp = jnp.concatenate([src, jnp.zeros((_EP - E,), jnp.int32)])
    dstp = jnp.concatenate([dst, jnp.full((_EP - E,), N, jnp.int32)])
    edges_cm = jnp.stack(
        [srcp.reshape(_NCH, _B), dstp.reshape(_NCH, _B)], axis=1)

    x1, at1 = _prep1(embp, W_src1, acat1)
    e1, ss1 = _edge_a1(at1, edges_cm, zeros8)
    acc1 = _edge_b1(x1.reshape(_NP * HEADS, C1), e1, zeros1)
    x2, at2 = _fin1(acc1, ss1, bias1.reshape(1, HID), W_src2, acat2)
    e2, ss2 = _edge_a2(at2, edges_cm, zeros8)
    acc2 = _edge_b2(x2, e2, zeros2)
    h2 = _fin2(acc2.reshape(2, _NA, OUT), ss2, bias2.reshape(1, OUT))
    return h2


# speedup vs baseline: 74.1082x; 1.1793x over previous
import jax
import jax.numpy as jnp
from jax import lax
from jax.experimental import pallas as pl
from jax.experimental.pallas import tpu as pltpu
from jax.experimental.pallas import tpu_sc as plsc

N = 50000
E = 800000
HID = 64
OUT = 32
HEADS = 4

_B = 128
_NW = 32
_KPW = 196
_NCH = _NW * _KPW
_EP = _NCH * _B
_NA = 50048
_RPT = _NA // 16
_NP = 50176

_SC_PARAMS = pltpu.CompilerParams(
    needs_layout_passes=False, use_tc_tiling_on_sc=False)
_MESH = dict(core_axis_name="c", subcore_axis_name="s")


def _make_edge_a(H):
    G = (H + 1) // 2
    P = H // G

    def body(at_hbm, edges_hbm, zeros_hbm, e_hbm, ssum_hbm,
             eb0, eb1, ats0, ats1, atd0, atd1, ebuf, stage, ssum_sh,
             ss0, ss1, sd0, sd1):
        cid = lax.axis_index("c")
        sid = lax.axis_index("s")
        w = sid * 2 + cid
        r0 = sid * _RPT
        eb = (eb0, eb1)
        ats = (ats0, ats1)
        atd = (atd0, atd1)
        ssem = (ss0, ss1)
        dsem = (sd0, sd1)

        z16 = jnp.zeros((16,), jnp.float32)
        for j in range(_B // 16):
            ridx0 = lax.iota(jnp.int32, 16) + (j * 16)
            for c in range(H, 8):
                plsc.store_scatter(
                    stage, [ridx0, jnp.full((16,), c, jnp.int32)], z16)

        pltpu.sync_copy(zeros_hbm.at[pl.ds(r0, _RPT)],
                        ssum_sh.at[pl.ds(r0, _RPT)])
        plsc.subcore_barrier()

        for b in (0, 1):
            pltpu.sync_copy(edges_hbm.at[w + _NW * b], eb[b])
            pltpu.async_copy(at_hbm.at[eb[b].at[0]], ats[b], ssem[b])
            pltpu.async_copy(at_hbm.at[eb[b].at[1]], atd[b], dsem[b])

        def pair(i, carry):
            for b in (0, 1):
                k = 2 * i + b
                c = w + _NW * k
                pltpu.make_async_copy(
                    at_hbm.at[eb[b].at[0]], ats[b], ssem[b]).wait()
                pltpu.make_async_copy(
                    at_hbm.at[eb[b].at[1]], atd[b], dsem[b]).wait()
                for j in range(_B // 16):
                    sl = pl.ds(j * 16, 16)
                    ridx = lax.iota(jnp.int32, 16) + (j * 16)
                    s16 = eb[b][0, sl]
                    d16 = eb[b][1, sl]
                    for h in range(H):
                        av = plsc.load_gather(
                            ats[b], [ridx, jnp.full((16,), h, jnp.int32)])
                        bv = plsc.load_gather(
                            atd[b], [ridx, jnp.full((16,), 4 + h, jnp.int32)])
                        s = av + bv
                        e = jnp.exp(jnp.where(s >= 0, s, 0.2 * s))
                        ebuf[h // P, h % P, sl] = plsc.bitcast(e, jnp.int32)
                        plsc.store_scatter(
                            stage, [ridx, jnp.full((16,), h, jnp.int32)], e)
                    for g in range(G):
                        ebuf[g, P, sl] = s16 * G + g
                        ebuf[g, P + 1, sl] = d16
                pltpu.sync_copy(ebuf, e_hbm.at[c])
                pltpu.sync_copy(stage, ssum_sh.at[eb[b].at[1]], add=True)
                kn = jnp.minimum(k + 2, _KPW - 1)
                pltpu.sync_copy(edges_hbm.at[w + _NW * kn], eb[b])
                pltpu.async_copy(at_hbm.at[eb[b].at[0]], ats[b], ssem[b])
                pltpu.async_copy(at_hbm.at[eb[b].at[1]], atd[b], dsem[b])
            return carry

        lax.fori_loop(0, _KPW // 2, pair, 0)
        for b in (0, 1):
            pltpu.make_async_copy(
                at_hbm.at[eb[b].at[0]], ats[b], ssem[b]).wait()
            pltpu.make_async_copy(
                at_hbm.at[eb[b].at[1]], atd[b], dsem[b]).wait()
        plsc.subcore_barrier()
        pltpu.sync_copy(ssum_sh.at[pl.ds(r0, _RPT)],
                        ssum_hbm.at[cid, pl.ds(r0, _RPT)])
        plsc.subcore_barrier()

    return pl.kernel(
        body,
        out_type=(
            jax.ShapeDtypeStruct((_NCH, G, P + 2, _B), jnp.int32),
            jax.ShapeDtypeStruct((2, _NA, 8), jnp.float32),
        ),
        mesh=plsc.VectorSubcoreMesh(**_MESH),
        scratch_types=[
            pltpu.VMEM((2, _B), jnp.int32),
            pltpu.VMEM((2, _B), jnp.int32),
            pltpu.VMEM((_B, 8), jnp.float32),
            pltpu.VMEM((_B, 8), jnp.float32),
            pltpu.VMEM((_B, 8), jnp.float32),
            pltpu.VMEM((_B, 8), jnp.float32),
            pltpu.VMEM((G, P + 2, _B), jnp.int32),
            pltpu.VMEM((_B, 8), jnp.float32),
            pltpu.VMEM_SHARED((_NA, 8), jnp.float32),
            pltpu.SemaphoreType.DMA,
            pltpu.SemaphoreType.DMA,
            pltpu.SemaphoreType.DMA,
            pltpu.SemaphoreType.DMA,
        ],
        compiler_params=_SC_PARAMS,
    )


def _make_edge_b(H, C):
    G = (H + 1) // 2
    P = H // G
    CW = P * C

    def body(xv_hbm, rec_hbm, zeros_hbm, out_hbm,
             rb0, rb1, rw0, rw1, acc_sh, sg0, sg1):
        cid = lax.axis_index("c")
        sid = lax.axis_index("s")
        w = sid * 2 + cid
        r0 = sid * _RPT
        recb = (rb0, rb1)
        rows = (rw0, rw1)
        gsem = (sg0, sg1)

        def load_and_gather(b, k, g):
            c = w + _NW * k
            pltpu.sync_copy(rec_hbm.at[c, g], recb[b])
            pltpu.async_copy(xv_hbm.at[recb[b].at[P]], rows[b], gsem[b])

        def gwait(b):
            pltpu.make_async_copy(
                xv_hbm.at[recb[b].at[P]], rows[b], gsem[b]).wait()

        for g in range(G):
            pltpu.sync_copy(zeros_hbm.at[pl.ds(r0, _RPT)],
                            acc_sh.at[pl.ds(r0, _RPT)])
            plsc.subcore_barrier()

            for b in (0, 1):
                load_and_gather(b, b, g)

            def pair(i, carry):
                for b in (0, 1):
                    k = 2 * i + b
                    gwait(b)
                    for j in range(_B // 16):
                        sl = pl.ds(j * 16, 16)
                        e16s = [plsc.bitcast(recb[b][p, sl], jnp.float32)
                                for p in range(P)]
                        for i16 in range(16):
                            kk = j * 16 + i16
                            for p in range(P):
                                ev = e16s[p].at[
                                    jnp.full((16,), i16, jnp.int32)].get(
                                        mode="promise_in_bounds")
                                for c0 in range(0, C, 16):
                                    cc = p * C + c0
                                    rows[b][kk, pl.ds(cc, 16)] = (
                                        rows[b][kk, pl.ds(cc, 16)] * ev)
                    pltpu.sync_copy(rows[b], acc_sh.at[recb[b].at[P + 1]],
                                    add=True)
                    kn = jnp.minimum(k + 2, _KPW - 1)
                    load_and_gather(b, kn, g)
                return carry

            lax.fori_loop(0, _KPW // 2, pair, 0)
            for b in (0, 1):
                gwait(b)
            plsc.subcore_barrier()
            pltpu.sync_copy(acc_sh.at[pl.ds(r0, _RPT)],
                            out_hbm.at[g, cid, pl.ds(r0, _RPT)])
            plsc.subcore_barrier()

    return pl.kernel(
        body,
        out_type=jax.ShapeDtypeStruct((G, 2, _NA, CW), jnp.float32),
        mesh=plsc.VectorSubcoreMesh(**_MESH),
        scratch_types=[
            pltpu.VMEM((P + 2, _B), jnp.int32),
            pltpu.VMEM((P + 2, _B), jnp.int32),
            pltpu.VMEM((_B, CW), jnp.float32),
            pltpu.VMEM((_B, CW), jnp.float32),
            pltpu.VMEM_SHARED((_NA, CW), jnp.float32),
            pltpu.SemaphoreType.DMA,
            pltpu.SemaphoreType.DMA,
        ],
        compiler_params=_SC_PARAMS,
    )


_edge_a1 = _make_edge_a(HEADS)
_edge_a2 = _make_edge_a(1)
_edge_b1 = _make_edge_b(HEADS, HID // HEADS)
_edge_b2 = _make_edge_b(1, OUT)

_NB = 2000
_NB1 = 1792


def _prep1_body(h_ref, w_ref, acat_ref, x_ref, at_ref):
    hb = h_ref[...]
    x_ref[...] = jnp.dot(hb, w_ref[...], preferred_element_type=jnp.float32)
    at_ref[...] = jnp.dot(hb, acat_ref[...],
                          preferred_element_type=jnp.float32)


_prep1 = pl.pallas_call(
    _prep1_body,
    grid=(_NP // _NB1,),
    in_specs=[
        pl.BlockSpec((_NB1, HID), lambda i: (i, 0)),
        pl.BlockSpec((HID, HID), lambda i: (0, 0)),
        pl.BlockSpec((HID, 8), lambda i: (0, 0)),
    ],
    out_specs=[
        pl.BlockSpec((_NB1, HID), lambda i: (i, 0)),
        pl.BlockSpec((_NB1, 8), lambda i: (i, 0)),
    ],
    out_shape=[
        jax.ShapeDtypeStruct((_NP, HID), jnp.float32),
        jax.ShapeDtypeStruct((_NP, 8), jnp.float32),
    ],
)


def _fin1_body(acc_ref, ss_ref, b1_ref, w2_ref, acat2_ref, x2_ref, at2_ref):
    acc = acc_ref[...]
    a = acc[:, 0] + acc[:, 1]
    ss = ss_ref[...]
    s = ss[0] + ss[1]
    cols = []
    for h in range(HEADS):
        g, off = divmod(h, 2)
        off *= 16
        cols.append(a[g, :, off:off + 16] / (s[:, h:h + 1] + 1e-16))
    h1 = jnp.concatenate(cols, axis=1) + b1_ref[...]
    h1 = jnp.maximum(h1, 0.0)
    x2_ref[...] = jnp.dot(h1, w2_ref[...], preferred_element_type=jnp.float32)
    at2_ref[...] = jnp.dot(h1, acat2_ref[...],
                           preferred_element_type=jnp.float32)


_fin1 = pl.pallas_call(
    _fin1_body,
    grid=(N // _NB,),
    in_specs=[
        pl.BlockSpec((2, 2, _NB, 32), lambda i: (0, 0, i, 0)),
        pl.BlockSpec((2, _NB, 8), lambda i: (0, i, 0)),
        pl.BlockSpec((1, HID), lambda i: (0, 0)),
        pl.BlockSpec((HID, OUT), lambda i: (0, 0)),
        pl.BlockSpec((HID, 8), lambda i: (0, 0)),
    ],
    out_specs=[
        pl.BlockSpec((_NB, OUT), lambda i: (i, 0)),
        pl.BlockSpec((_NB, 8), lambda i: (i, 0)),
    ],
    out_shape=[
        jax.ShapeDtypeStruct((N, OUT), jnp.float32),
        jax.ShapeDtypeStruct((N, 8), jnp.float32),
    ],
)


def _fin2_body(acc_ref, ss_ref, b2_ref, out_ref):
    acc = acc_ref[...]
    a = acc[0] + acc[1]
    ss = ss_ref[...]
    s = ss[0] + ss[1]
    out_ref[...] = jnp.maximum(
        a / (s[:, 0:1] + 1e-16) + b2_ref[...], 0.0)


_fin2 = pl.pallas_call(
    _fin2_body,
    grid=(N // _NB,),
    in_specs=[
        pl.BlockSpec((2, _NB, OUT), lambda i: (0, i, 0)),
        pl.BlockSpec((2, _NB, 8), lambda i: (0, i, 0)),
        pl.BlockSpec((1, OUT), lambda i: (0, 0)),
    ],
    out_specs=pl.BlockSpec((_NB, OUT), lambda i: (i, 0)),
    out_shape=jax.ShapeDtypeStruct((N, OUT), jnp.float32),
)


def kernel(x, edge_index, emb, W_src1, W_dst1, att_src1, att_dst1, bias1,
           W_src2, W_dst2, att_src2, att_dst2, bias2):
    del x
    src = edge_index[0]
    dst = edge_index[1]
    C1 = HID // HEADS

    As1 = (W_src1.reshape(HID, HEADS, C1) * att_src1).sum(-1)
    Ad1 = (W_dst1.reshape(HID, HEADS, C1) * att_dst1).sum(-1)
    acat1 = jnp.concatenate([As1, Ad1], axis=1)
    As2 = (W_src2.reshape(HID, 1, OUT) * att_src2).sum(-1)
    Ad2 = (W_dst2.reshape(HID, 1, OUT) * att_dst2).sum(-1)
    pad3 = jnp.zeros((HID, 3), jnp.float32)
    acat2 = jnp.concatenate([As2, pad3, Ad2, pad3], axis=1)

    zeros8 = jnp.zeros((_NA, 8), jnp.float32)
    zeros1 = jnp.zeros((_NA, 2 * C1), jnp.float32)
    zeros2 = jnp.zeros((_NA, OUT), jnp.float32)
    embp = jnp.pad(emb, ((0, _NP - N), (0, 0)))

    srcp = jnp.concatenate([src, jnp.zeros((_EP - E,), jnp.int32)])
    dstp = jnp.concatenate([dst, jnp.full((_EP - E,), N, jnp.int32)])
    edges_cm = jnp.stack(
        [srcp.reshape(_NCH, _B), dstp.reshape(_NCH, _B)], axis=1)

    x1, at1 = _prep1(embp, W_src1, acat1)
    e1, ss1 = _edge_a1(at1, edges_cm, zeros8)
    acc1 = _edge_b1(x1.reshape(_NP * 2, 2 * C1), e1, zeros1)
    x2, at2 = _fin1(acc1, ss1, bias1.reshape(1, HID), W_src2, acat2)
    e2, ss2 = _edge_a2(at2, edges_cm, zeros8)
    acc2 = _edge_b2(x2, e2, zeros2)
    h2 = _fin2(acc2.reshape(2, _NA, OUT), ss2, bias2.reshape(1, OUT))
    return h2


# speedup vs baseline: 77.3975x; 1.0444x over previous
import jax
import jax.numpy as jnp
from jax import lax
from jax.experimental import pallas as pl
from jax.experimental.pallas import tpu as pltpu
from jax.experimental.pallas import tpu_sc as plsc

N = 50000
E = 800000
HID = 64
OUT = 32
HEADS = 4

_B = 128
_NW = 32
_KPW = 196
_NCH = _NW * _KPW
_EP = _NCH * _B
_NA = 50048
_RPT = _NA // 16
_NP = 50176

_SC_PARAMS = pltpu.CompilerParams(
    needs_layout_passes=False, use_tc_tiling_on_sc=False)
_MESH = dict(core_axis_name="c", subcore_axis_name="s")


def _make_edge_a(H):
    G = (H + 1) // 2
    P = H // G

    def body(at_hbm, edges_hbm, zeros_hbm, e_hbm, ssum_hbm,
             eb0, eb1, ats0, ats1, atd0, atd1, ebuf, stage, ssum_sh,
             ss0, ss1, sd0, sd1):
        cid = lax.axis_index("c")
        sid = lax.axis_index("s")
        w = sid * 2 + cid
        r0 = sid * _RPT
        eb = (eb0, eb1)
        ats = (ats0, ats1)
        atd = (atd0, atd1)
        ssem = (ss0, ss1)
        dsem = (sd0, sd1)

        z16 = jnp.zeros((16,), jnp.float32)
        for j in range(_B // 16):
            ridx0 = lax.iota(jnp.int32, 16) + (j * 16)
            for c in range(H, 8):
                plsc.store_scatter(
                    stage, [ridx0, jnp.full((16,), c, jnp.int32)], z16)

        pltpu.sync_copy(zeros_hbm.at[pl.ds(r0, _RPT)],
                        ssum_sh.at[pl.ds(r0, _RPT)])
        plsc.subcore_barrier()

        for b in (0, 1):
            pltpu.sync_copy(edges_hbm.at[w + _NW * b], eb[b])
            pltpu.async_copy(at_hbm.at[eb[b].at[0]], ats[b], ssem[b])
            pltpu.async_copy(at_hbm.at[eb[b].at[1]], atd[b], dsem[b])

        def pair(i, carry):
            for b in (0, 1):
                k = 2 * i + b
                c = w + _NW * k
                pltpu.make_async_copy(
                    at_hbm.at[eb[b].at[0]], ats[b], ssem[b]).wait()
                pltpu.make_async_copy(
                    at_hbm.at[eb[b].at[1]], atd[b], dsem[b]).wait()
                for j in range(_B // 16):
                    sl = pl.ds(j * 16, 16)
                    ridx = lax.iota(jnp.int32, 16) + (j * 16)
                    s16 = eb[b][0, sl]
                    d16 = eb[b][1, sl]
                    for h in range(H):
                        av = plsc.load_gather(
                            ats[b], [ridx, jnp.full((16,), h, jnp.int32)])
                        bv = plsc.load_gather(
                            atd[b], [ridx, jnp.full((16,), 4 + h, jnp.int32)])
                        s = av + bv
                        e = jnp.exp(jnp.where(s >= 0, s, 0.2 * s))
                        ebuf[h // P, h % P, sl] = plsc.bitcast(e, jnp.int32)
                        plsc.store_scatter(
                            stage, [ridx, jnp.full((16,), h, jnp.int32)], e)
                    for g in range(G):
                        ebuf[g, P, sl] = s16 * G + g
                        ebuf[g, P + 1, sl] = d16
                pltpu.sync_copy(ebuf, e_hbm.at[c])
                pltpu.sync_copy(stage, ssum_sh.at[eb[b].at[1]], add=True)
                kn = jnp.minimum(k + 2, _KPW - 1)
                pltpu.sync_copy(edges_hbm.at[w + _NW * kn], eb[b])
                pltpu.async_copy(at_hbm.at[eb[b].at[0]], ats[b], ssem[b])
                pltpu.async_copy(at_hbm.at[eb[b].at[1]], atd[b], dsem[b])
            return carry

        lax.fori_loop(0, _KPW // 2, pair, 0)
        for b in (0, 1):
            pltpu.make_async_copy(
                at_hbm.at[eb[b].at[0]], ats[b], ssem[b]).wait()
            pltpu.make_async_copy(
                at_hbm.at[eb[b].at[1]], atd[b], dsem[b]).wait()
        plsc.subcore_barrier()
        pltpu.sync_copy(ssum_sh.at[pl.ds(r0, _RPT)],
                        ssum_hbm.at[cid, pl.ds(r0, _RPT)])
        plsc.subcore_barrier()

    return pl.kernel(
        body,
        out_type=(
            jax.ShapeDtypeStruct((_NCH, G, P + 2, _B), jnp.int32),
            jax.ShapeDtypeStruct((2, _NA, 8), jnp.float32),
        ),
        mesh=plsc.VectorSubcoreMesh(**_MESH),
        scratch_types=[
            pltpu.VMEM((2, _B), jnp.int32),
            pltpu.VMEM((2, _B), jnp.int32),
            pltpu.VMEM((_B, 8), jnp.float32),
            pltpu.VMEM((_B, 8), jnp.float32),
            pltpu.VMEM((_B, 8), jnp.float32),
            pltpu.VMEM((_B, 8), jnp.float32),
            pltpu.VMEM((G, P + 2, _B), jnp.int32),
            pltpu.VMEM((_B, 8), jnp.float32),
            pltpu.VMEM_SHARED((_NA, 8), jnp.float32),
            pltpu.SemaphoreType.DMA,
            pltpu.SemaphoreType.DMA,
            pltpu.SemaphoreType.DMA,
            pltpu.SemaphoreType.DMA,
        ],
        compiler_params=_SC_PARAMS,
    )


def _make_edge_b(H, C):
    G = (H + 1) // 2
    P = H // G
    CW = P * C

    def body(xv_hbm, rec_hbm, zeros_hbm, out_hbm,
             rb0, rb1, dc0, dc1, rw0, rw1, rw2, rw3, acc_sh,
             sg0, sg1, sc0, sc1):
        cid = lax.axis_index("c")
        sid = lax.axis_index("s")
        w = sid * 2 + cid
        r0 = sid * _RPT
        recb = (rb0, rb1)
        dstc = (dc0, dc1)
        rows = (rw0, rw1, rw2, rw3)
        gsem = (sg0, sg1)
        scsem = (sc0, sc1)

        def load_and_gather(b, q, k, g):
            c = w + _NW * k
            pltpu.sync_copy(rec_hbm.at[c, g], recb[b])
            pltpu.async_copy(xv_hbm.at[recb[b].at[P]], rows[q], gsem[b])

        def step(b, q, k, g, swait):
            pltpu.make_async_copy(
                xv_hbm.at[recb[b].at[P]], rows[q], gsem[b]).wait()
            for j in range(_B // 16):
                sl = pl.ds(j * 16, 16)
                e16s = [plsc.bitcast(recb[b][p, sl], jnp.float32)
                        for p in range(P)]
                for i16 in range(16):
                    kk = j * 16 + i16
                    for p in range(P):
                        ev = e16s[p].at[
                            jnp.full((16,), i16, jnp.int32)].get(
                                mode="promise_in_bounds")
                        for c0 in range(0, C, 16):
                            cc = p * C + c0
                            rows[q][kk, pl.ds(cc, 16)] = (
                                rows[q][kk, pl.ds(cc, 16)] * ev)
            if swait:
                pltpu.make_async_copy(
                    rows[q ^ 2], acc_sh.at[dstc[b]], scsem[b]).wait()
            for j in range(_B // 16):
                sl = pl.ds(j * 16, 16)
                dstc[b][sl] = recb[b][P + 1, sl]
            pltpu.async_copy(rows[q], acc_sh.at[dstc[b]], scsem[b],
                             add=True)
            kn = jnp.minimum(k + 2, _KPW - 1)
            load_and_gather(b, q ^ 2, kn, g)

        for g in range(G):
            pltpu.sync_copy(zeros_hbm.at[pl.ds(r0, _RPT)],
                            acc_sh.at[pl.ds(r0, _RPT)])
            plsc.subcore_barrier()

            for b in (0, 1):
                load_and_gather(b, b, b, g)
            for b in (0, 1):
                step(b, b, b, g, swait=False)

            def pair_even(i, carry):
                for b in (0, 1):
                    step(b, b ^ 2, 4 * i + 2 + b, g, swait=True)
                for b in (0, 1):
                    step(b, b, 4 * i + 4 + b, g, swait=True)
                return carry

            lax.fori_loop(0, (_KPW - 2) // 4, pair_even, 0)
            for b in (0, 1):
                step(b, b ^ 2, _KPW - 2 + b, g, swait=True)
            for b in (0, 1):
                pltpu.make_async_copy(
                    xv_hbm.at[recb[b].at[P]], rows[b], gsem[b]).wait()
                pltpu.make_async_copy(
                    rows[b ^ 2], acc_sh.at[dstc[b]], scsem[b]).wait()
            plsc.subcore_barrier()
            pltpu.sync_copy(acc_sh.at[pl.ds(r0, _RPT)],
                            out_hbm.at[g, cid, pl.ds(r0, _RPT)])
            plsc.subcore_barrier()

    return pl.kernel(
        body,
        out_type=jax.ShapeDtypeStruct((G, 2, _NA, CW), jnp.float32),
        mesh=plsc.VectorSubcoreMesh(**_MESH),
        scratch_types=[
            pltpu.VMEM((P + 2, _B), jnp.int32),
            pltpu.VMEM((P + 2, _B), jnp.int32),
            pltpu.VMEM((_B,), jnp.int32),
            pltpu.VMEM((_B,), jnp.int32),
            pltpu.VMEM((_B, CW), jnp.float32),
            pltpu.VMEM((_B, CW), jnp.float32),
            pltpu.VMEM((_B, CW), jnp.float32),
            pltpu.VMEM((_B, CW), jnp.float32),
            pltpu.VMEM_SHARED((_NA, CW), jnp.float32),
            pltpu.SemaphoreType.DMA,
            pltpu.SemaphoreType.DMA,
            pltpu.SemaphoreType.DMA,
            pltpu.SemaphoreType.DMA,
        ],
        compiler_params=_SC_PARAMS,
    )


_edge_a1 = _make_edge_a(HEADS)
_edge_a2 = _make_edge_a(1)
_edge_b1 = _make_edge_b(HEADS, HID // HEADS)
_edge_b2 = _make_edge_b(1, OUT)

_NB = 2000
_NB1 = 1792


def _prep1_body(h_ref, w_ref, acat_ref, x_ref, at_ref):
    hb = h_ref[...]
    x_ref[...] = jnp.dot(hb, w_ref[...], preferred_element_type=jnp.float32)
    at_ref[...] = jnp.dot(hb, acat_ref[...],
                          preferred_element_type=jnp.float32)


_prep1 = pl.pallas_call(
    _prep1_body,
    grid=(_NP // _NB1,),
    in_specs=[
        pl.BlockSpec((_NB1, HID), lambda i: (i, 0)),
        pl.BlockSpec((HID, HID), lambda i: (0, 0)),
        pl.BlockSpec((HID, 8), lambda i: (0, 0)),
    ],
    out_specs=[
        pl.BlockSpec((_NB1, HID), lambda i: (i, 0)),
        pl.BlockSpec((_NB1, 8), lambda i: (i, 0)),
    ],
    out_shape=[
        jax.ShapeDtypeStruct((_NP, HID), jnp.float32),
        jax.ShapeDtypeStruct((_NP, 8), jnp.float32),
    ],
)


def _fin1_body(acc_ref, ss_ref, b1_ref, w2_ref, acat2_ref, x2_ref, at2_ref):
    acc = acc_ref[...]
    a = acc[:, 0] + acc[:, 1]
    ss = ss_ref[...]
    s = ss[0] + ss[1]
    cols = []
    for h in range(HEADS):
        g, off = divmod(h, 2)
        off *= 16
        cols.append(a[g, :, off:off + 16] / (s[:, h:h + 1] + 1e-16))
    h1 = jnp.concatenate(cols, axis=1) + b1_ref[...]
    h1 = jnp.maximum(h1, 0.0)
    x2_ref[...] = jnp.dot(h1, w2_ref[...], preferred_element_type=jnp.float32)
    at2_ref[...] = jnp.dot(h1, acat2_ref[...],
                           preferred_element_type=jnp.float32)


_fin1 = pl.pallas_call(
    _fin1_body,
    grid=(N // _NB,),
    in_specs=[
        pl.BlockSpec((2, 2, _NB, 32), lambda i: (0, 0, i, 0)),
        pl.BlockSpec((2, _NB, 8), lambda i: (0, i, 0)),
        pl.BlockSpec((1, HID), lambda i: (0, 0)),
        pl.BlockSpec((HID, OUT), lambda i: (0, 0)),
        pl.BlockSpec((HID, 8), lambda i: (0, 0)),
    ],
    out_specs=[
        pl.BlockSpec((_NB, OUT), lambda i: (i, 0)),
        pl.BlockSpec((_NB, 8), lambda i: (i, 0)),
    ],
    out_shape=[
        jax.ShapeDtypeStruct((N, OUT), jnp.float32),
        jax.ShapeDtypeStruct((N, 8), jnp.float32),
    ],
)


def _fin2_body(acc_ref, ss_ref, b2_ref, out_ref):
    acc = acc_ref[...]
    a = acc[0] + acc[1]
    ss = ss_ref[...]
    s = ss[0] + ss[1]
    out_ref[...] = jnp.maximum(
        a / (s[:, 0:1] + 1e-16) + b2_ref[...], 0.0)


_fin2 = pl.pallas_call(
    _fin2_body,
    grid=(N // _NB,),
    in_specs=[
        pl.BlockSpec((2, _NB, OUT), lambda i: (0, i, 0)),
        pl.BlockSpec((2, _NB, 8), lambda i: (0, i, 0)),
        pl.BlockSpec((1, OUT), lambda i: (0, 0)),
    ],
    out_specs=pl.BlockSpec((_NB, OUT), lambda i: (i, 0)),
    out_shape=jax.ShapeDtypeStruct((N, OUT), jnp.float32),
)


def kernel(x, edge_index, emb, W_src1, W_dst1, att_src1, att_dst1, bias1,
           W_src2, W_dst2, att_src2, att_dst2, bias2):
    del x
    src = edge_index[0]
    dst = edge_index[1]
    C1 = HID // HEADS

    As1 = (W_src1.reshape(HID, HEADS, C1) * att_src1).sum(-1)
    Ad1 = (W_dst1.reshape(HID, HEADS, C1) * att_dst1).sum(-1)
    acat1 = jnp.concatenate([As1, Ad1], axis=1)
    As2 = (W_src2.reshape(HID, 1, OUT) * att_src2).sum(-1)
    Ad2 = (W_dst2.reshape(HID, 1, OUT) * att_dst2).sum(-1)
    pad3 = jnp.zeros((HID, 3), jnp.float32)
    acat2 = jnp.concatenate([As2, pad3, Ad2, pad3], axis=1)

    zeros8 = jnp.zeros((_NA, 8), jnp.float32)
    zeros1 = jnp.zeros((_NA, 2 * C1), jnp.float32)
    zeros2 = jnp.zeros((_NA, OUT), jnp.float32)
    embp = jnp.pad(emb, ((0, _NP - N), (0, 0)))

    srcp = jnp.concatenate([src, jnp.zeros((_EP - E,), jnp.int32)])
    dstp = jnp.concatenate([dst, jnp.full((_EP - E,), N, jnp.int32)])
    edges_cm = jnp.stack(
        [srcp.reshape(_NCH, _B), dstp.reshape(_NCH, _B)], axis=1)

    x1, at1 = _prep1(embp, W_src1, acat1)
    e1, ss1 = _edge_a1(at1, edges_cm, zeros8)
    acc1 = _edge_b1(x1.reshape(_NP * 2, 2 * C1), e1, zeros1)
    x2, at2 = _fin1(acc1, ss1, bias1.reshape(1, HID), W_src2, acat2)
    e2, ss2 = _edge_a2(at2, edges_cm, zeros8)
    acc2 = _edge_b2(x2, e2, zeros2)
    h2 = _fin2(acc2.reshape(2, _NA, OUT), ss2, bias2.reshape(1, OUT))
    return h2
